# baseline (speedup 1.0000x reference)
import functools

import jax
import jax.numpy as jnp
from jax import lax
from jax.experimental import pallas as pl
from jax.experimental.pallas import tpu as pltpu
from jax.experimental.pallas import tpu_sc as plsc

_NC, _NS, _L = 2, 16, 16
_NW = _NC * _NS
_B = 16384
_D = 32
_BPW = _B // _NW

_sc_mesh = plsc.VectorSubcoreMesh(core_axis_name="c", subcore_axis_name="s")


@functools.partial(
    pl.kernel,
    out_type=(
        jax.ShapeDtypeStruct((_B, _D), jnp.float32),
        jax.ShapeDtypeStruct((_B, _D), jnp.float32),
    ),
    mesh=_sc_mesh,
    scratch_types=[
        pltpu.VMEM((_BPW,), jnp.int32),
        pltpu.VMEM((_BPW,), jnp.int32),
        pltpu.VMEM((_BPW, _D), jnp.float32),
        pltpu.VMEM((_BPW, _D), jnp.float32),
        pltpu.SemaphoreType.DMA,
        pltpu.SemaphoreType.DMA,
    ],
    compiler_params=pltpu.CompilerParams(use_tc_tiling_on_sc=False),
)
def _sc_gather(user_hbm, movie_hbm, ue_hbm, me_hbm, uout_hbm, mout_hbm,
               uidx_v, midx_v, urows_v, mrows_v, usem, msem):
    wid = lax.axis_index("s") * _NC + lax.axis_index("c")
    base = wid * _BPW
    pltpu.sync_copy(user_hbm.at[pl.ds(base, _BPW)], uidx_v)
    pltpu.sync_copy(movie_hbm.at[pl.ds(base, _BPW)], midx_v)
    ucp = pltpu.async_copy(ue_hbm.at[uidx_v], urows_v, usem)
    mcp = pltpu.async_copy(me_hbm.at[midx_v], mrows_v, msem)
    ucp.wait()
    mcp.wait()
    pltpu.sync_copy(urows_v, uout_hbm.at[pl.ds(base, _BPW)])
    pltpu.sync_copy(mrows_v, mout_hbm.at[pl.ds(base, _BPW)])


def _mlp_body(u_ref, m_ref, w1u_ref, w1m_ref, b1_ref, w2_ref, b2_ref,
              w3_ref, b3_ref, out_ref):
    h = jnp.dot(u_ref[...], w1u_ref[...], preferred_element_type=jnp.float32)
    h = h + jnp.dot(m_ref[...], w1m_ref[...],
                    preferred_element_type=jnp.float32)
    h = jnp.maximum(h + b1_ref[...], 0.0)
    h = jnp.dot(h, w2_ref[...], preferred_element_type=jnp.float32)
    h = jnp.maximum(h + b2_ref[...], 0.0)
    o = jnp.dot(h, w3_ref[...], preferred_element_type=jnp.float32)
    out_ref[...] = o + b3_ref[...]


def kernel(user, movie, user_emb, movie_emb, W1, b1, W2, b2, W3, b3):
    u_rows, m_rows = _sc_gather(user.astype(jnp.int32),
                                movie.astype(jnp.int32),
                                user_emb, movie_emb)
    out = pl.pallas_call(
        _mlp_body,
        out_shape=jax.ShapeDtypeStruct((_B, 1), jnp.float32),
    )(u_rows, m_rows,
      W1[:, :_D].T, W1[:, _D:].T, b1.reshape(1, 64),
      W2.T, b2.reshape(1, 32),
      W3.T, b3.reshape(1, 1))
    return out.reshape(_B)


# speedup vs baseline: 1.4613x; 1.4613x over previous
import functools

import jax
import jax.numpy as jnp
from jax import lax
from jax.experimental import pallas as pl
from jax.experimental.pallas import tpu as pltpu
from jax.experimental.pallas import tpu_sc as plsc

_NC, _NS, _L = 2, 16, 16
_NW = _NC * _NS
_B = 16384
_D = 32
_BPW = _B // _NW
_K = 8

_sc_mesh = plsc.VectorSubcoreMesh(core_axis_name="c", subcore_axis_name="s")


@functools.partial(
    pl.kernel,
    out_type=jax.ShapeDtypeStruct((_B, _D), jnp.float32),
    mesh=_sc_mesh,
    scratch_types=[
        pltpu.VMEM((_BPW,), jnp.int32),
        pltpu.VMEM((_BPW, _D), jnp.float32),
        pltpu.SemaphoreType.DMA,
    ],
)
def _sc_rowgather(idx_hbm, table_hbm, out_hbm, idx_v, rows_v, sem):
    wid = lax.axis_index("s") * _NC + lax.axis_index("c")
    base = wid * _BPW
    pltpu.sync_copy(idx_hbm.at[pl.ds(base, _BPW)], idx_v)

    def group(g, _):
        b0 = g * _L
        vec = idx_v[pl.ds(b0, _L)]
        cps = []
        for j in range(_L):
            cps.append(pltpu.async_copy(
                table_hbm.at[pl.ds(vec[j], 1)], rows_v.at[pl.ds(b0 + j, 1)],
                sem))
        for cp in cps:
            cp.wait()
        return 0

    lax.fori_loop(0, _BPW // _L, group, 0)
    pltpu.sync_copy(rows_v, out_hbm.at[pl.ds(base, _BPW)])


def _mlp_body(u_ref, m_ref, w1u_ref, w1m_ref, b1_ref, w2_ref, b2_ref,
              w3_ref, b3_ref, out_ref):
    h = jnp.dot(u_ref[...], w1u_ref[...], preferred_element_type=jnp.float32)
    h = h + jnp.dot(m_ref[...], w1m_ref[...],
                    preferred_element_type=jnp.float32)
    h = jnp.maximum(h + b1_ref[...], 0.0)
    h = jnp.dot(h, w2_ref[...], preferred_element_type=jnp.float32)
    h = jnp.maximum(h + b2_ref[...], 0.0)
    o = jnp.dot(h, w3_ref[...], preferred_element_type=jnp.float32)
    out_ref[...] = o + b3_ref[...]


def kernel(user, movie, user_emb, movie_emb, W1, b1, W2, b2, W3, b3):
    u_rows = _sc_rowgather(user.astype(jnp.int32), user_emb)
    m_rows = _sc_rowgather(movie.astype(jnp.int32), movie_emb)
    out = pl.pallas_call(
        _mlp_body,
        out_shape=jax.ShapeDtypeStruct((_B, 1), jnp.float32),
    )(u_rows, m_rows,
      W1[:, :_D].T, W1[:, _D:].T, b1.reshape(1, 64),
      W2.T, b2.reshape(1, 32),
      W3.T, b3.reshape(1, 1))
    return out.reshape(_B)
